# baseline (speedup 1.0000x reference)
import functools

import jax
import jax.numpy as jnp
from jax import lax
from jax.experimental import pallas as pl
from jax.experimental.pallas import tpu as pltpu
from jax.experimental.pallas import tpu_sc as plsc

MAXLEN = 200
EMB = 64
NUM_TILES = 32
NBUF = 4
SPLIT = (104, 96)


def _tok_pos_embed(x_flat, token_table, pos_table):
    total = x_flat.shape[0]
    rows_per_tile = total // NUM_TILES
    nchunk = rows_per_tile // MAXLEN
    mesh = plsc.VectorSubcoreMesh(core_axis_name="c", subcore_axis_name="s")

    @functools.partial(
        pl.kernel,
        out_type=jax.ShapeDtypeStruct((total, EMB), jnp.float32),
        mesh=mesh,
        compiler_params=pltpu.CompilerParams(use_tc_tiling_on_sc=False),
        scratch_types=[
            pltpu.VMEM((rows_per_tile,), jnp.int32),
            pltpu.VMEM((MAXLEN, EMB), jnp.float32),
        ] + [pltpu.VMEM((MAXLEN, EMB), jnp.float32) for _ in range(NBUF)]
          + [pltpu.SemaphoreType.DMA for _ in range(2 * NBUF)],
    )
    def k(x_hbm, tok_hbm, pos_hbm, out_hbm, idx_v, pos_v, *bufs_and_sems):
        bufs = bufs_and_sems[:NBUF]
        gsems = bufs_and_sems[NBUF:2 * NBUF]
        osems = bufs_and_sems[2 * NBUF:]
        wid = lax.axis_index("s") * 2 + lax.axis_index("c")
        base = wid * rows_per_tile
        pltpu.sync_copy(x_hbm.at[pl.ds(base, rows_per_tile)], idx_v)
        pltpu.sync_copy(pos_hbm, pos_v)

        def issue_gather(c, b):
            off = c * MAXLEN
            r0 = 0
            for n in SPLIT:
                pltpu.async_copy(
                    tok_hbm.at[idx_v.at[pl.ds(off + r0, n)]],
                    bufs[b].at[pl.ds(r0, n)], gsems[b])
                r0 += n

        def wait_gather(c, b):
            off = c * MAXLEN
            r0 = 0
            for n in SPLIT:
                pltpu.make_async_copy(
                    tok_hbm.at[idx_v.at[pl.ds(off + r0, n)]],
                    bufs[b].at[pl.ds(r0, n)], gsems[b]).wait()
                r0 += n

        def issue_out(c, b):
            pltpu.async_copy(bufs[b], out_hbm.at[pl.ds(base + c * MAXLEN, MAXLEN)],
                             osems[b])

        def wait_out(c, b):
            pltpu.make_async_copy(bufs[b],
                                  out_hbm.at[pl.ds(base + c * MAXLEN, MAXLEN)],
                                  osems[b]).wait()

        issue_gather(0, 0)
        issue_gather(1, 1)

        @pl.loop(0, nchunk, step=NBUF)
        def _grp(g):
            for b in range(NBUF):
                c = g + b
                bp = (b + 2) % NBUF
                wait_gather(c, b)

                @pl.when(c + 2 < nchunk)
                def _prefetch():
                    @pl.when(c >= 2)
                    def _drain():
                        wait_out(c - 2, bp)
                    issue_gather(c + 2, bp)

                if False:
                    @pl.loop(0, MAXLEN, unroll=4)
                    def _row(r):
                        for col in range(0, EMB, 16):
                            plsc.addupdate(bufs[b].at[r, pl.ds(col, 16)],
                                           pos_v[r, pl.ds(col, 16)])

                issue_out(c, b)

        for b in range(NBUF):
            wait_out(nchunk - NBUF + b, b)

    return k(x_flat, token_table, pos_table)


def kernel(x, token_table, pos_table):
    batch, seq = x.shape
    if seq < MAXLEN:
        x = jnp.pad(x, ((0, 0), (0, MAXLEN - seq)))
    else:
        x = x[:, :MAXLEN]
    x_flat = x.reshape(-1).astype(jnp.int32)
    out = _tok_pos_embed(x_flat, token_table, pos_table)
    return out.reshape(batch, MAXLEN, EMB)


# speedup vs baseline: 1.2258x; 1.2258x over previous
import functools

import jax
import jax.numpy as jnp
from jax import lax
from jax.experimental import pallas as pl
from jax.experimental.pallas import tpu as pltpu
from jax.experimental.pallas import tpu_sc as plsc

MAXLEN = 200
EMB = 64
LANES = 128
NUM_TILES = 32
NBUF = 4
SPLIT = (104, 96)


def _tok_pos_embed(x_flat, tok_padded, pos_table):
    total = x_flat.shape[0]
    rows_per_tile = total // NUM_TILES
    nchunk = 2 * (rows_per_tile // MAXLEN)
    mesh = plsc.VectorSubcoreMesh(core_axis_name="c", subcore_axis_name="s")

    @functools.partial(
        pl.kernel,
        out_type=jax.ShapeDtypeStruct((total, LANES), jnp.float32),
        mesh=mesh,
        scratch_types=[
            pltpu.VMEM((rows_per_tile,), jnp.int32),
            pltpu.VMEM((MAXLEN, EMB), jnp.float32),
        ] + [pltpu.VMEM((SPLIT[0], LANES), jnp.float32) for _ in range(NBUF)]
          + [pltpu.SemaphoreType.DMA for _ in range(2 * NBUF)],
    )
    def k(x_hbm, tok_hbm, pos_hbm, out_hbm, idx_v, pos_v, *bufs_and_sems):
        bufs = bufs_and_sems[:NBUF]
        gsems = bufs_and_sems[NBUF:2 * NBUF]
        osems = bufs_and_sems[2 * NBUF:]
        wid = lax.axis_index("s") * 2 + lax.axis_index("c")
        base = wid * rows_per_tile
        pltpu.sync_copy(x_hbm.at[pl.ds(base, rows_per_tile)], idx_v)
        pltpu.sync_copy(pos_hbm, pos_v)

        def chunk_off(c):
            return (c // 2) * MAXLEN + (c % 2) * SPLIT[0]

        def issue_gather(c, b, n):
            pltpu.async_copy(
                tok_hbm.at[idx_v.at[pl.ds(chunk_off(c), n)]],
                bufs[b].at[pl.ds(0, n)], gsems[b])

        def wait_gather(c, b, n):
            pltpu.make_async_copy(
                tok_hbm.at[idx_v.at[pl.ds(chunk_off(c), n)]],
                bufs[b].at[pl.ds(0, n)], gsems[b]).wait()

        def issue_out(c, b, n):
            pltpu.async_copy(
                bufs[b].at[pl.ds(0, n)],
                out_hbm.at[pl.ds(base + chunk_off(c), n)], osems[b])

        def wait_out(c, b, n):
            pltpu.make_async_copy(
                bufs[b].at[pl.ds(0, n)],
                out_hbm.at[pl.ds(base + chunk_off(c), n)], osems[b]).wait()

        issue_gather(0, 0, SPLIT[0])
        issue_gather(1, 1, SPLIT[1])

        @pl.loop(0, nchunk, step=NBUF)
        def _grp(g):
            for b in range(NBUF):
                c = g + b
                n = SPLIT[b % 2]
                p0 = (b % 2) * SPLIT[0]
                bp = (b + 2) % NBUF
                np_ = SPLIT[bp % 2]
                wait_gather(c, b, n)

                @pl.when(c + 2 < nchunk)
                def _prefetch():
                    @pl.when(c >= 2)
                    def _drain():
                        wait_out(c - 2, bp, np_)
                    issue_gather(c + 2, bp, np_)

                @pl.loop(0, n, unroll=4)
                def _row(r):
                    for col in range(0, EMB, 16):
                        plsc.addupdate(bufs[b].at[r, pl.ds(col, 16)],
                                       pos_v[p0 + r, pl.ds(col, 16)])

                issue_out(c, b, n)

        for b in range(NBUF):
            wait_out(nchunk - NBUF + b, b, SPLIT[b % 2])

    return k(x_flat, tok_padded, pos_table)


def kernel(x, token_table, pos_table):
    batch, seq = x.shape
    if seq < MAXLEN:
        x = jnp.pad(x, ((0, 0), (0, MAXLEN - seq)))
    else:
        x = x[:, :MAXLEN]
    x_flat = x.reshape(-1).astype(jnp.int32)
    tok_padded = jnp.pad(token_table, ((0, 0), (0, LANES - EMB)))
    out = _tok_pos_embed(x_flat, tok_padded, pos_table)
    return out[:, :EMB].reshape(batch, MAXLEN, EMB)
